# Initial kernel scaffold; baseline (speedup 1.0000x reference)
import jax
import jax.numpy as jnp
from jax.experimental import pallas as pl


def kernel(x, edge_index, W_l1, b_l1, W_r1, W_l2, b_l2, W_r2):
    raise NotImplementedError("write your pallas kernel here")


# speedup vs baseline: 3.3329x; 3.3329x over previous
import functools

import jax
import jax.numpy as jnp
from jax import lax
from jax.experimental import pallas as pl
from jax.experimental.pallas import tpu as pltpu
from jax.experimental.pallas import tpu_sc as plsc

N = 10000
D = 128
H = D // 2
E = 320000
WID = H + 16
ACC_ROWS = 10240
NC, NS = 2, 16
CHUNK = 128
TILE_CH = 160
E_PAD = NS * TILE_CH * CHUNK
ROWS_PER_TILE = ACC_ROWS // NS

_BLK = 400
_GRID = N // _BLK

_f32 = jnp.float32
_HIGH = jax.lax.Precision.HIGHEST


def _dot(a, b):
    return jnp.dot(a, b, precision=_HIGH, preferred_element_type=_f32)


def _tables(xw):
    ones = jnp.ones((_BLK, WID - H), _f32)
    return (jnp.concatenate([xw[:, :H], ones], axis=1),
            jnp.concatenate([xw[:, H:], ones], axis=1))


def _pre_body(x_ref, wl_ref, wr_ref, t0_ref, t1_ref, xr_ref):
    xb = x_ref[...]
    t0, t1 = _tables(_dot(xb, wl_ref[...]))
    t0_ref[...] = t0
    t1_ref[...] = t1
    xr_ref[...] = _dot(xb, wr_ref[...])


def _combine(p0_ref, p1_ref, b_ref, add_ref):
    p0 = p0_ref[...]
    p1 = p1_ref[...]
    seg = jnp.concatenate([p0[:, :H], p1[:, :H]], axis=1)
    cnt = p0[:, H:H + 1]
    mean = seg / jnp.maximum(cnt, 1.0)
    return mean + b_ref[...] + add_ref[...]


def _mid_body(p0_ref, p1_ref, b_ref, xr_ref, wl_ref, wr_ref,
              hid_ref, t0_ref, t1_ref, hr_ref):
    hid = _combine(p0_ref, p1_ref, b_ref, xr_ref)
    hid_ref[...] = hid
    t0, t1 = _tables(_dot(hid, wl_ref[...]))
    t0_ref[...] = t0
    t1_ref[...] = t1
    hr_ref[...] = _dot(hid, wr_ref[...])


def _post_body(p0_ref, p1_ref, b_ref, hr_ref, out_ref):
    out_ref[...] = _combine(p0_ref, p1_ref, b_ref, hr_ref)


_row_spec = pl.BlockSpec((_BLK, D), lambda i: (i, 0))
_half_spec = pl.BlockSpec((_BLK, WID), lambda i: (i, 0))
_w_spec = pl.BlockSpec((D, D), lambda i: (0, 0))
_b_spec = pl.BlockSpec((1, D), lambda i: (0, 0))

_pre_call = pl.pallas_call(
    _pre_body,
    grid=(_GRID,),
    in_specs=[_row_spec, _w_spec, _w_spec],
    out_specs=[_half_spec, _half_spec, _row_spec],
    out_shape=[jax.ShapeDtypeStruct((N, WID), _f32),
               jax.ShapeDtypeStruct((N, WID), _f32),
               jax.ShapeDtypeStruct((N, D), _f32)],
)

_mid_call = pl.pallas_call(
    _mid_body,
    grid=(_GRID,),
    in_specs=[_half_spec, _half_spec, _b_spec, _row_spec, _w_spec, _w_spec],
    out_specs=[_row_spec, _half_spec, _half_spec, _row_spec],
    out_shape=[jax.ShapeDtypeStruct((N, D), _f32),
               jax.ShapeDtypeStruct((N, WID), _f32),
               jax.ShapeDtypeStruct((N, WID), _f32),
               jax.ShapeDtypeStruct((N, D), _f32)],
)

_post_call = pl.pallas_call(
    _post_body,
    grid=(_GRID,),
    in_specs=[_half_spec, _half_spec, _b_spec, _row_spec],
    out_specs=_row_spec,
    out_shape=jax.ShapeDtypeStruct((N, D), _f32),
)


def _sc_seg_body(table_hbm, src_hbm, dst_hbm, zeros_hbm, out_hbm,
                 src_v, dst_v, rows0, rows1, acc, sem0, sem1):
    c = lax.axis_index("c")
    s = lax.axis_index("s")

    pltpu.sync_copy(src_hbm.at[c * NS + s], src_v)
    pltpu.sync_copy(dst_hbm.at[s], dst_v)

    r0 = s * ROWS_PER_TILE
    pltpu.sync_copy(zeros_hbm.at[pl.ds(r0, ROWS_PER_TILE)],
                    acc.at[pl.ds(r0, ROWS_PER_TILE)])
    plsc.subcore_barrier()

    @pl.loop(0, TILE_CH, step=2)
    def _(j):
        cp0 = pltpu.async_copy(table_hbm.at[src_v.at[j]], rows0, sem0)
        cp1 = pltpu.async_copy(table_hbm.at[src_v.at[j + 1]], rows1, sem1)
        cp0.wait()
        pltpu.sync_copy(rows0, acc.at[dst_v.at[j]], add=True)
        cp1.wait()
        pltpu.sync_copy(rows1, acc.at[dst_v.at[j + 1]], add=True)

    plsc.subcore_barrier()
    pltpu.sync_copy(acc.at[pl.ds(r0, ROWS_PER_TILE)],
                    out_hbm.at[c].at[pl.ds(r0, ROWS_PER_TILE)])


_sc_seg = functools.partial(
    pl.kernel,
    out_type=jax.ShapeDtypeStruct((NC, ACC_ROWS, WID), _f32),
    mesh=plsc.VectorSubcoreMesh(core_axis_name="c", subcore_axis_name="s"),
    compiler_params=pltpu.CompilerParams(use_tc_tiling_on_sc=False),
    scratch_types=[
        pltpu.VMEM((TILE_CH, CHUNK), jnp.int32),
        pltpu.VMEM((TILE_CH, CHUNK), jnp.int32),
        pltpu.VMEM((CHUNK, WID), _f32),
        pltpu.VMEM((CHUNK, WID), _f32),
        pltpu.VMEM_SHARED((ACC_ROWS, WID), _f32),
        pltpu.SemaphoreType.DMA,
        pltpu.SemaphoreType.DMA,
    ],
)(_sc_seg_body)


@jax.jit
def _run(x, edge_index, W_l1, b_l1, W_r1, W_l2, b_l2, W_r2):
    src = edge_index[0]
    dst = edge_index[1]
    pad = E_PAD - E
    src_p = jnp.concatenate([src, jnp.zeros((pad,), jnp.int32)])
    src_b = jnp.concatenate([src_p, src_p + N]).reshape(
        NC * NS, TILE_CH, CHUNK)
    dst_b = jnp.concatenate(
        [dst, jnp.full((pad,), N, jnp.int32)]).reshape(NS, TILE_CH, CHUNK)
    zeros = jnp.zeros((ACC_ROWS, WID), _f32)
    b1 = b_l1.reshape(1, D)
    b2 = b_l2.reshape(1, D)

    t0, t1, xr1 = _pre_call(x, W_l1, W_r1)
    table1 = jnp.concatenate([t0, t1], axis=0)
    parts1 = _sc_seg(table1, src_b, dst_b, zeros)
    hid, t0, t1, hr2 = _mid_call(parts1[0], parts1[1], b1, xr1, W_l2, W_r2)
    table2 = jnp.concatenate([t0, t1], axis=0)
    parts2 = _sc_seg(table2, src_b, dst_b, zeros)
    out = _post_call(parts2[0], parts2[1], b2, hr2)
    return out, hid


def kernel(x, edge_index, W_l1, b_l1, W_r1, W_l2, b_l2, W_r2):
    return _run(x, edge_index, W_l1, b_l1, W_r1, W_l2, b_l2, W_r2)
